# Initial kernel scaffold; baseline (speedup 1.0000x reference)
import jax
import jax.numpy as jnp
from jax.experimental import pallas as pl


def kernel(logits):
    raise NotImplementedError("write your pallas kernel here")


# speedup vs baseline: 9.8503x; 9.8503x over previous
import jax
import jax.numpy as jnp
from jax import lax
from jax.experimental import pallas as pl
from jax.experimental.pallas import tpu as pltpu
from jax.experimental.pallas import tpu_sc as plsc

TOP_P = 0.9
ROWS = 128
VOCAB = 100000
LANES = 16
CHUNKS = VOCAB // LANES
NB = 1024
HIST_WORDS = LANES * NB

_NC = 2
_NS = 16
ROWS_PER_WORKER = ROWS // (_NC * _NS)


def _kernel_body(logits_hbm, out_hbm, row_v, hist_v):
    lane = lax.iota(jnp.int32, LANES)
    wid = lax.axis_index("s") * _NC + lax.axis_index("c")

    def monokey(x):
        u = lax.bitcast_convert_type(x, jnp.uint32)
        s = u >> jnp.uint32(31)
        m = (jnp.uint32(0) - s) | jnp.uint32(0x80000000)
        return u ^ m

    def inv_monokey(key):
        kt = key >> jnp.uint32(31)
        mi = (kt - jnp.uint32(1)) | jnp.uint32(0x80000000)
        return lax.bitcast_convert_type(key ^ mi, jnp.float32)

    def zero_hist(nwords):
        def zb(j, c):
            hist_v[pl.ds(j * LANES, LANES)] = jnp.zeros((LANES,), jnp.float32)
            return c

        lax.fori_loop(0, nwords // LANES, zb, 0)

    def search(num_vregs, lane_stride, a_init, tau):
        def it(jj, carry):
            b_sel, a_sel, acc = carry
            j = num_vregs - 1 - jj
            hv = hist_v[pl.ds(j * LANES, LANES)]
            for l in range(1, LANES):
                hv = hv + hist_v[pl.ds(l * lane_stride + j * LANES, LANES)]
            ioa = jnp.flip(jnp.cumsum(jnp.flip(hv)))
            sa = ioa - hv
            s_vec = acc + sa
            cond = (s_vec <= tau) & (hv > jnp.float32(0))
            anyc = jnp.any(cond)
            lane_sel = jnp.max(plsc.all_reduce_ffs(cond))
            sa_sel = jnp.sum(jnp.where(lane == lane_sel, sa, jnp.float32(0)))
            b_sel = jnp.where(anyc, j * LANES + lane_sel, b_sel)
            a_sel = jnp.where(anyc, acc + sa_sel, a_sel)
            return b_sel, a_sel, acc + jnp.sum(hv)

        b_sel, a_sel, _ = lax.fori_loop(
            0, num_vregs, it, (jnp.int32(0), jnp.float32(0), a_init)
        )
        return b_sel, a_sel

    def hist_pass(shift, nbits, lane_stride, prefix_bits, prefix):
        bmask = jnp.uint32((1 << nbits) - 1)

        def it(i, c):
            x = row_v[pl.ds(i * LANES, LANES)]
            key = monokey(x)
            e = jnp.exp(x)
            b = ((key >> jnp.uint32(shift)) & bmask).astype(jnp.int32)
            idx = lane * lane_stride + b
            if prefix_bits == 0:
                plsc.addupdate_scatter(hist_v, [idx], e)
            else:
                match = (key >> jnp.uint32(32 - prefix_bits)) == prefix
                plsc.addupdate_scatter(hist_v, [idx], e, mask=match)
            return c

        lax.fori_loop(0, CHUNKS, it, 0)

    def row_body(rr, carry):
        r = wid * ROWS_PER_WORKER + rr
        pltpu.sync_copy(logits_hbm.at[r], row_v)

        zero_hist(HIST_WORDS)
        hist_pass(22, 10, NB, 0, jnp.uint32(0))

        def zsum(j, acc):
            return acc + hist_v[pl.ds(j * LANES, LANES)]

        z_vec = lax.fori_loop(0, HIST_WORDS // LANES, zsum,
                              jnp.zeros((LANES,), jnp.float32))
        z_total = jnp.sum(z_vec)
        tau = jnp.float32(TOP_P) * z_total

        b1, a1 = search(NB // LANES, NB, jnp.float32(0), tau)

        p1 = b1.astype(jnp.uint32)
        zero_hist(HIST_WORDS)
        hist_pass(12, 10, NB, 10, p1)
        b2, a2 = search(NB // LANES, NB, a1, tau)

        p2 = (p1 << jnp.uint32(10)) | b2.astype(jnp.uint32)
        zero_hist(HIST_WORDS)
        hist_pass(2, 10, NB, 20, p2)
        b3, a3 = search(NB // LANES, NB, a2, tau)

        p3 = (p2 << jnp.uint32(10)) | b3.astype(jnp.uint32)
        zero_hist(LANES * LANES)
        hist_pass(0, 2, LANES, 30, p3)
        b4, b_above = search(1, LANES, a3, tau)

        key_star = (p3 << jnp.uint32(2)) | b4.astype(jnp.uint32)
        e_star = jnp.exp(inv_monokey(jnp.full((LANES,), key_star)))
        e_star_s = jnp.max(e_star)

        def mask_it(i, cnt):
            x = row_v[pl.ds(i * LANES, LANES)]
            key = monokey(x)
            eq = key == key_star
            gt = key > key_star
            eqi = eq.astype(jnp.int32)
            rank = cnt + plsc.cumsum(eqi) - eqi
            keep_eq = (b_above + rank.astype(jnp.float32) * e_star_s) <= tau
            kept = gt | (eq & keep_eq)
            row_v[pl.ds(i * LANES, LANES)] = jnp.where(
                kept, x, jnp.float32(-jnp.inf)
            )
            return cnt + jnp.sum(eqi)

        lax.fori_loop(0, CHUNKS, mask_it, jnp.int32(0))

        pltpu.sync_copy(row_v, out_hbm.at[r])
        return carry

    lax.fori_loop(0, ROWS_PER_WORKER, row_body, 0)


@jax.jit
def kernel(logits):
    mesh = plsc.VectorSubcoreMesh(core_axis_name="c", subcore_axis_name="s")
    run = pl.kernel(
        _kernel_body,
        out_type=jax.ShapeDtypeStruct((ROWS, VOCAB), jnp.float32),
        mesh=mesh,
        scratch_types=[
            pltpu.VMEM((VOCAB,), jnp.float32),
            pltpu.VMEM((HIST_WORDS,), jnp.float32),
        ],
        compiler_params=pltpu.CompilerParams(needs_layout_passes=False),
    )
    return run(logits)


# speedup vs baseline: 12.7296x; 1.2923x over previous
import jax
import jax.numpy as jnp
from jax import lax
from jax.experimental import pallas as pl
from jax.experimental.pallas import tpu as pltpu
from jax.experimental.pallas import tpu_sc as plsc

TOP_P = 0.9
ROWS = 128
VOCAB = 100000
LANES = 16
CHUNKS = VOCAB // LANES
NB = 1024
HIST_WORDS = LANES * NB

_NC = 2
_NS = 16
ROWS_PER_WORKER = ROWS // (_NC * _NS)


def _kernel_body(logits_hbm, out_hbm, row_v, hist_v):
    lane = lax.iota(jnp.int32, LANES)
    wid = lax.axis_index("s") * _NC + lax.axis_index("c")

    def monokey(x):
        u = lax.bitcast_convert_type(x, jnp.uint32)
        s = u >> jnp.uint32(31)
        m = (jnp.uint32(0) - s) | jnp.uint32(0x80000000)
        return u ^ m

    def inv_monokey(key):
        kt = key >> jnp.uint32(31)
        mi = (kt - jnp.uint32(1)) | jnp.uint32(0x80000000)
        return lax.bitcast_convert_type(key ^ mi, jnp.float32)

    def zero_hist(nwords):
        zu = 16 if nwords >= 256 * LANES else 1
        zvec = jnp.zeros((LANES,), jnp.float32)

        def zb(j, c):
            for k in range(zu):
                hist_v[pl.ds((j * zu + k) * LANES, LANES)] = zvec
            return c

        lax.fori_loop(0, nwords // (LANES * zu), zb, 0)

    def search(num_vregs, lane_stride, a_init, tau):
        def it(jj, carry):
            b_sel, a_sel, acc = carry
            j = num_vregs - 1 - jj
            hv = hist_v[pl.ds(j * LANES, LANES)]
            for l in range(1, LANES):
                hv = hv + hist_v[pl.ds(l * lane_stride + j * LANES, LANES)]
            ioa = jnp.flip(jnp.cumsum(jnp.flip(hv)))
            sa = ioa - hv
            s_vec = acc + sa
            cond = (s_vec <= tau) & (hv > jnp.float32(0))
            anyc = jnp.any(cond)
            lane_sel = jnp.max(plsc.all_reduce_ffs(cond))
            sa_sel = jnp.sum(jnp.where(lane == lane_sel, sa, jnp.float32(0)))
            b_sel = jnp.where(anyc, j * LANES + lane_sel, b_sel)
            a_sel = jnp.where(anyc, acc + sa_sel, a_sel)
            return b_sel, a_sel, acc + jnp.sum(hv)

        b_sel, a_sel, _ = lax.fori_loop(
            0, num_vregs, it, (jnp.int32(0), jnp.float32(0), a_init)
        )
        return b_sel, a_sel

    def hist_pass(shift, nbits, lane_stride, prefix_bits, prefix):
        bmask = jnp.uint32((1 << nbits) - 1)
        u = 10

        def it(ii, c):
            for k in range(u):
                i = ii * u + k
                x = row_v[pl.ds(i * LANES, LANES)]
                key = monokey(x)
                e = jnp.exp(x)
                b = ((key >> jnp.uint32(shift)) & bmask).astype(jnp.int32)
                idx = lane * lane_stride + b
                if prefix_bits == 0:
                    plsc.addupdate_scatter(hist_v, [idx], e)
                else:
                    match = (key >> jnp.uint32(32 - prefix_bits)) == prefix
                    plsc.addupdate_scatter(hist_v, [idx], e, mask=match)
            return c

        lax.fori_loop(0, CHUNKS // u, it, 0)

    def row_body(rr, carry):
        r = wid * ROWS_PER_WORKER + rr
        pltpu.sync_copy(logits_hbm.at[r], row_v)

        zero_hist(HIST_WORDS)
        hist_pass(22, 10, NB, 0, jnp.uint32(0))

        def zsum(j, acc):
            vs = [hist_v[pl.ds((j * 16 + k) * LANES, LANES)] for k in range(16)]
            while len(vs) > 1:
                vs = [vs[m] + vs[m + 1] for m in range(0, len(vs), 2)]
            return acc + vs[0]

        z_vec = lax.fori_loop(0, HIST_WORDS // (LANES * 16), zsum,
                              jnp.zeros((LANES,), jnp.float32))
        z_total = jnp.sum(z_vec)
        tau = jnp.float32(TOP_P) * z_total

        b1, a1 = search(NB // LANES, NB, jnp.float32(0), tau)

        p1 = b1.astype(jnp.uint32)
        zero_hist(HIST_WORDS)
        hist_pass(12, 10, NB, 10, p1)
        b2, a2 = search(NB // LANES, NB, a1, tau)

        p2 = (p1 << jnp.uint32(10)) | b2.astype(jnp.uint32)
        zero_hist(HIST_WORDS)
        hist_pass(2, 10, NB, 20, p2)
        b3, a3 = search(NB // LANES, NB, a2, tau)

        p3 = (p2 << jnp.uint32(10)) | b3.astype(jnp.uint32)
        zero_hist(LANES * LANES)
        hist_pass(0, 2, LANES, 30, p3)
        b4, b_above = search(1, LANES, a3, tau)

        key_star = (p3 << jnp.uint32(2)) | b4.astype(jnp.uint32)
        e_star = jnp.exp(inv_monokey(jnp.full((LANES,), key_star)))
        e_star_s = jnp.max(e_star)

        mu = 10

        def mask_it(ii, cnt):
            for k in range(mu):
                i = ii * mu + k
                x = row_v[pl.ds(i * LANES, LANES)]
                key = monokey(x)
                eq = key == key_star
                gt = key > key_star
                eqi = eq.astype(jnp.int32)
                rank = cnt + plsc.cumsum(eqi) - eqi
                keep_eq = (b_above + rank.astype(jnp.float32) * e_star_s) <= tau
                kept = gt | (eq & keep_eq)
                row_v[pl.ds(i * LANES, LANES)] = jnp.where(
                    kept, x, jnp.float32(-jnp.inf)
                )
                cnt = cnt + jnp.sum(eqi)
            return cnt

        lax.fori_loop(0, CHUNKS // mu, mask_it, jnp.int32(0))

        pltpu.sync_copy(row_v, out_hbm.at[r])
        return carry

    lax.fori_loop(0, ROWS_PER_WORKER, row_body, 0)


@jax.jit
def kernel(logits):
    mesh = plsc.VectorSubcoreMesh(core_axis_name="c", subcore_axis_name="s")
    run = pl.kernel(
        _kernel_body,
        out_type=jax.ShapeDtypeStruct((ROWS, VOCAB), jnp.float32),
        mesh=mesh,
        scratch_types=[
            pltpu.VMEM((VOCAB,), jnp.float32),
            pltpu.VMEM((HIST_WORDS,), jnp.float32),
        ],
        compiler_params=pltpu.CompilerParams(needs_layout_passes=False),
    )
    return run(logits)
